# Initial kernel scaffold; baseline (speedup 1.0000x reference)
import jax
import jax.numpy as jnp
from jax.experimental import pallas as pl


def kernel(text, table, W, b):
    raise NotImplementedError("write your pallas kernel here")


# speedup vs baseline: 1.8496x; 1.8496x over previous
import functools

import jax
import jax.numpy as jnp
from jax import lax
from jax.experimental import pallas as pl
from jax.experimental.pallas import tpu as pltpu
from jax.experimental.pallas import tpu_sc as plsc

_NC = 2
_NS = 16
_NW = _NC * _NS
_LANES = 16

_DIM = 64
_HALF = 100
_CHB = 32


def _sc_embed_sum(text2, table):
  hb, half = text2.shape
  assert half == _HALF
  nbags = hb // 2
  hb_per_w = hb // _NW
  assert hb_per_w % _CHB == 0
  n_chunks = hb_per_w // _CHB
  ncol = _DIM // _LANES

  mesh = plsc.VectorSubcoreMesh(core_axis_name="c", subcore_axis_name="s")

  @functools.partial(
      pl.kernel,
      mesh=mesh,
      compiler_params=pltpu.CompilerParams(use_tc_tiling_on_sc=False),
      out_type=jax.ShapeDtypeStruct((nbags, _DIM), jnp.float32),
      scratch_types=[
          pltpu.VMEM((_CHB, _HALF), jnp.int32),
          pltpu.VMEM((_HALF, _DIM), jnp.float32),
          pltpu.VMEM((_CHB // 2, _DIM), jnp.float32),
          pltpu.SemaphoreType.DMA,
      ],
  )
  def k(text2_hbm, table_hbm, out_hbm, idx_v, rows_v, sums_v, sem):
    wid = lax.axis_index("s") * _NC + lax.axis_index("c")
    hb_base = wid * hb_per_w

    def chunk_body(ci, _):
      chunk_hb = pl.multiple_of(hb_base + ci * _CHB, _CHB)
      pltpu.sync_copy(text2_hbm.at[pl.ds(chunk_hb, _CHB)], idx_v)

      def bag_body(t, _):
        accs = tuple(jnp.zeros((_LANES,), jnp.float32) for _ in range(ncol))
        for h in range(2):
          pltpu.async_copy(
              table_hbm.at[idx_v.at[2 * t + h]], rows_v, sem
          ).wait()

          def row_body(i, a):
            return tuple(
                a[c] + rows_v[i, pl.ds(c * _LANES, _LANES)]
                for c in range(ncol)
            )

          accs = lax.fori_loop(0, _HALF, row_body, accs)
        for c in range(ncol):
          sums_v[t, pl.ds(c * _LANES, _LANES)] = accs[c]
        return 0

      lax.fori_loop(0, _CHB // 2, bag_body, 0)
      pltpu.sync_copy(
          sums_v,
          out_hbm.at[pl.ds(pl.multiple_of(chunk_hb // 2, _CHB // 2), _CHB // 2)],
      )
      return 0

    lax.fori_loop(0, n_chunks, chunk_body, 0)

  return k(text2, table)


def _tc_linear(summed, text, wt, bias):
  nbags, seq = text.shape
  out_dim = wt.shape[1]
  blk = 1024
  grid = (nbags // blk,)

  def body(summed_ref, text_ref, wt_ref, b_ref, out_ref):
    cnt = jnp.sum(
        (text_ref[...] != 0).astype(jnp.float32), axis=1, keepdims=True
    )
    pooled = summed_ref[...] / jnp.maximum(cnt, 1.0)
    out_ref[...] = (
        jnp.dot(pooled, wt_ref[...], preferred_element_type=jnp.float32)
        + b_ref[...]
    )

  return pl.pallas_call(
      body,
      grid=grid,
      in_specs=[
          pl.BlockSpec((blk, _DIM), lambda i: (i, 0)),
          pl.BlockSpec((blk, seq), lambda i: (i, 0)),
          pl.BlockSpec((_DIM, out_dim), lambda i: (0, 0)),
          pl.BlockSpec((1, out_dim), lambda i: (0, 0)),
      ],
      out_specs=pl.BlockSpec((blk, out_dim), lambda i: (i, 0)),
      out_shape=jax.ShapeDtypeStruct((nbags, out_dim), jnp.float32),
  )(summed, text, wt, bias)


def kernel(text, table, W, b):
  nbags, seq = text.shape
  text = text.astype(jnp.int32)
  text2 = text.reshape(nbags * 2, seq // 2)
  summed = _sc_embed_sum(text2, table)
  return _tc_linear(summed, text, W.T, b.reshape(1, -1))


# speedup vs baseline: 3.5410x; 1.9145x over previous
import functools

import jax
import jax.numpy as jnp
from jax import lax
from jax.experimental import pallas as pl
from jax.experimental.pallas import tpu as pltpu
from jax.experimental.pallas import tpu_sc as plsc

_NC = 2
_NS = 16
_NW = _NC * _NS
_LANES = 16

_DIM = 64
_HALF = 100
_CHB = 128
_NBUF = 4


def _sc_embed_sum(text2, table):
  hb, half = text2.shape
  assert half == _HALF
  nbags = hb // 2
  hb_per_w = hb // _NW
  assert hb_per_w % _CHB == 0
  n_chunks = hb_per_w // _CHB
  ncol = _DIM // _LANES

  mesh = plsc.VectorSubcoreMesh(core_axis_name="c", subcore_axis_name="s")

  @functools.partial(
      pl.kernel,
      mesh=mesh,
      compiler_params=pltpu.CompilerParams(use_tc_tiling_on_sc=False),
      out_type=jax.ShapeDtypeStruct((nbags, _DIM), jnp.float32),
      scratch_types=[
          pltpu.VMEM((_CHB, _HALF), jnp.int32),
          [pltpu.VMEM((_HALF, _DIM), jnp.float32) for _ in range(_NBUF)],
          pltpu.VMEM((_CHB // 2, _DIM), jnp.float32),
          [pltpu.SemaphoreType.DMA for _ in range(_NBUF)],
      ],
  )
  def k(text2_hbm, table_hbm, out_hbm, idx_v, rows, sums_v, sems):
    wid = lax.axis_index("s") * _NC + lax.axis_index("c")
    hb_base = wid * hb_per_w

    def gather(j, u):
      return pltpu.make_async_copy(table_hbm.at[idx_v.at[j]], rows[u], sems[u])

    def accum(u, t_local):
      def row_body(i, a):
        accs = list(a)
        for r in range(5):
          for c in range(ncol):
            accs[c] = accs[c] + rows[u][i * 5 + r, pl.ds(c * _LANES, _LANES)]
        return tuple(accs)

      return lax.fori_loop(
          0, _HALF // 5, row_body,
          tuple(t_local[c] for c in range(ncol)),
      )

    def chunk_body(ci, _):
      chunk_hb = pl.multiple_of(hb_base + ci * _CHB, _CHB)
      pltpu.sync_copy(text2_hbm.at[pl.ds(chunk_hb, _CHB)], idx_v)
      for u in range(_NBUF):
        gather(u, u).start()

      def group_body(g, _):
        for bag in range(2):
          accs = tuple(jnp.zeros((_LANES,), jnp.float32) for _ in range(ncol))
          for h in range(2):
            u = 2 * bag + h
            j = _NBUF * g + u
            gather(j, u).wait()
            accs = accum(u, accs)

            @pl.when(j + _NBUF < _CHB)
            def _():
              gather(j + _NBUF, u).start()

          t = 2 * g + bag
          for c in range(ncol):
            sums_v[t, pl.ds(c * _LANES, _LANES)] = accs[c]
        return 0

      lax.fori_loop(0, _CHB // _NBUF, group_body, 0)
      pltpu.sync_copy(
          sums_v,
          out_hbm.at[pl.ds(pl.multiple_of(chunk_hb // 2, _CHB // 2), _CHB // 2)],
      )
      return 0

    lax.fori_loop(0, n_chunks, chunk_body, 0)

  return k(text2, table)


def _tc_linear(summed, text, wt, bias):
  nbags, seq = text.shape
  out_dim = wt.shape[1]
  blk = 1024
  grid = (nbags // blk,)

  def body(summed_ref, text_ref, wt_ref, b_ref, out_ref):
    cnt = jnp.sum(
        (text_ref[...] != 0).astype(jnp.float32), axis=1, keepdims=True
    )
    pooled = summed_ref[...] / jnp.maximum(cnt, 1.0)
    out_ref[...] = (
        jnp.dot(pooled, wt_ref[...], preferred_element_type=jnp.float32)
        + b_ref[...]
    )

  return pl.pallas_call(
      body,
      grid=grid,
      in_specs=[
          pl.BlockSpec((blk, _DIM), lambda i: (i, 0)),
          pl.BlockSpec((blk, seq), lambda i: (i, 0)),
          pl.BlockSpec((_DIM, out_dim), lambda i: (0, 0)),
          pl.BlockSpec((1, out_dim), lambda i: (0, 0)),
      ],
      out_specs=pl.BlockSpec((blk, out_dim), lambda i: (i, 0)),
      out_shape=jax.ShapeDtypeStruct((nbags, out_dim), jnp.float32),
  )(summed, text, wt, bias)


def kernel(text, table, W, b):
  nbags, seq = text.shape
  text = text.astype(jnp.int32)
  text2 = text.reshape(nbags * 2, seq // 2)
  summed = _sc_embed_sum(text2, table)
  return _tc_linear(summed, text, W.T, b.reshape(1, -1))
